# baseline (speedup 1.0000x reference)
import functools

import jax
import jax.numpy as jnp
from jax import lax
from jax.experimental import pallas as pl
from jax.experimental.pallas import tpu as pltpu
from jax.experimental.pallas import tpu_sc as plsc

NC = 2
NS = 16
NW = NC * NS
LB = 128


@functools.lru_cache(maxsize=None)
def _build(batch, hist, feat):
    nbt = batch // LB
    bo_per_w = LB // NW
    nch = hist * bo_per_w
    per_w = bo_per_w * nbt * hist
    nft = feat // 8
    mesh = plsc.VectorSubcoreMesh(
        core_axis_name="c", subcore_axis_name="s", num_cores=NC, num_subcores=NS
    )
    out_sds = jax.ShapeDtypeStruct((hist, nft, 4, LB, 2 * nbt), jnp.bfloat16)
    buf_t = pltpu.VMEM((nbt, feat), jnp.float32)
    cidx_t = pltpu.VMEM((nbt,), jnp.int32)
    ob_t = pltpu.VMEM((nft, 4, bo_per_w, 2 * nbt), jnp.bfloat16)

    @functools.partial(
        pl.kernel,
        out_type=(out_sds, out_sds),
        mesh=mesh,
        compiler_params=pltpu.CompilerParams(
            needs_layout_passes=False, use_tc_tiling_on_sc=False
        ),
        scratch_types=[
            pltpu.VMEM((nbt, bo_per_w, hist), jnp.int32),
            (cidx_t, cidx_t),
            (buf_t, buf_t),
            (buf_t, buf_t),
            ob_t,
            ob_t,
            (pltpu.SemaphoreType.DMA, pltpu.SemaphoreType.DMA),
            (pltpu.SemaphoreType.DMA, pltpu.SemaphoreType.DMA),
        ],
    )
    def grab(x_hbm, real_hbm, imag_hbm, out_r_hbm, out_i_hbm,
             idx_v, cidx, buf_r, buf_i, ob_r, ob_i, sem_r, sem_i):
        wid = lax.axis_index("s") * NC + lax.axis_index("c")
        bo0 = wid * bo_per_w
        pltpu.sync_copy(x_hbm.at[:, pl.ds(bo0, bo_per_w)], idx_v)

        i16 = lax.iota(jnp.int32, 16)
        ridx = [i16 + bb * 16 for bb in range(8)]

        def build_cidx(c, p):
            h = c // bo_per_w
            bol = c - h * bo_per_w
            bv = jnp.full((16,), bol, jnp.int32)
            hv = jnp.full((16,), h, jnp.int32)
            for bb in range(8):
                cidx[p][pl.ds(bb * 16, 16)] = plsc.load_gather(
                    idx_v, [ridx[bb], bv, hv]
                )

        def start(p):
            pltpu.async_copy(real_hbm.at[cidx[p]], buf_r[p], sem_r[p])
            pltpu.async_copy(imag_hbm.at[cidx[p]], buf_i[p], sem_i[p])

        def wait(p):
            pltpu.make_async_copy(
                real_hbm.at[cidx[p]], buf_r[p], sem_r[p]).wait()
            pltpu.make_async_copy(
                imag_hbm.at[cidx[p]], buf_i[p], sem_i[p]).wait()

        def convert(buf, ob, bol):
            for ft in range(nft):
                for sw in range(4):
                    c0 = jnp.full((16,), ft * 8 + 2 * sw, jnp.int32)
                    c1 = c0 + 1
                    for bb in range(8):
                        f0 = plsc.load_gather(buf, [ridx[bb], c0])
                        f1 = plsc.load_gather(buf, [ridx[bb], c1])
                        ob[ft, sw, bol, pl.ds(bb * 32, 32)] = plsc.pack(
                            f0, f1, format=plsc.PackFormat.INTERLEAVED
                        )

        build_cidx(0, 0)
        start(0)

        @pl.loop(0, nch, step=2)
        def _chunk(j):
            for p in range(2):
                c = j + p
                q = 1 - p

                @pl.when(c + 1 < nch)
                def _():
                    build_cidx(c + 1, q)
                    start(q)

                wait(p)
                h = c // bo_per_w
                bol = c - h * bo_per_w
                convert(buf_r[p], ob_r, bol)
                convert(buf_i[p], ob_i, bol)

                @pl.when(bol == bo_per_w - 1)
                def _():
                    for ft in range(nft):
                        for sw in range(4):
                            pltpu.sync_copy(
                                ob_r.at[ft, sw],
                                out_r_hbm.at[h, ft, sw, pl.ds(bo0, bo_per_w)])
                            pltpu.sync_copy(
                                ob_i.at[ft, sw],
                                out_i_hbm.at[h, ft, sw, pl.ds(bo0, bo_per_w)])

    return grab


def kernel(x, real_embed, imag_embed):
    batch, hist = x.shape
    feat = real_embed.shape[1]
    x3 = x.reshape(batch // LB, LB, hist)
    fn = _build(batch, hist, feat)
    out_r, out_i = fn(x3, real_embed, imag_embed)

    def to_logical(o):
        o = o.reshape(hist, feat // 8, 4, LB, batch // LB, 2)
        return o.transpose(4, 3, 0, 1, 2, 5).reshape(batch, hist, feat)

    return to_logical(out_r), to_logical(out_i)


# speedup vs baseline: 1.0489x; 1.0489x over previous
import functools

import jax
import jax.numpy as jnp
from jax import lax
from jax.experimental import pallas as pl
from jax.experimental.pallas import tpu as pltpu
from jax.experimental.pallas import tpu_sc as plsc

NC = 2
NS = 16
NW = NC * NS
CH = 128


@functools.lru_cache(maxsize=None)
def _build(batch, hist, feat):
    bw = batch // NW
    ng = bw // CH
    per_w = bw * hist
    mesh = plsc.VectorSubcoreMesh(
        core_axis_name="c", subcore_axis_name="s", num_cores=NC, num_subcores=NS
    )
    out_sds = jax.ShapeDtypeStruct((hist * feat, batch), jnp.bfloat16)
    buf_t = pltpu.VMEM((bw, feat), jnp.float32)
    cidx_t = pltpu.VMEM((ng, CH), jnp.int32)
    zb_t = pltpu.VMEM((feat, bw), jnp.bfloat16)

    @functools.partial(
        pl.kernel,
        out_type=(out_sds, out_sds),
        mesh=mesh,
        compiler_params=pltpu.CompilerParams(
            needs_layout_passes=False, use_tc_tiling_on_sc=False
        ),
        scratch_types=[
            pltpu.VMEM((per_w,), jnp.int32),
            (cidx_t, cidx_t),
            (buf_t, buf_t),
            (buf_t, buf_t),
            zb_t,
            zb_t,
            (pltpu.SemaphoreType.DMA, pltpu.SemaphoreType.DMA),
            (pltpu.SemaphoreType.DMA, pltpu.SemaphoreType.DMA),
        ],
    )
    def grab(x_hbm, real_hbm, imag_hbm, out_r_hbm, out_i_hbm,
             idx_v, cidx, buf_r, buf_i, zb_r, zb_i, sem_r, sem_i):
        wid = lax.axis_index("s") * NC + lax.axis_index("c")
        b0 = wid * bw
        pltpu.sync_copy(x_hbm.at[pl.ds(b0 * hist, per_w)], idx_v)

        i16 = lax.iota(jnp.int32, 16)
        i16h = i16 * hist
        ev = [i16 * 2 + bb * 32 for bb in range(bw // 32)]

        def build_cidx(h, p):
            for g in range(ng):
                for bb in range(CH // 16):
                    base = (g * CH + bb * 16) * hist + h
                    cidx[p][g, pl.ds(bb * 16, 16)] = plsc.load_gather(
                        idx_v, [i16h + base]
                    )

        def start(p):
            for g in range(ng):
                pltpu.async_copy(real_hbm.at[cidx[p].at[g]],
                                 buf_r[p].at[pl.ds(g * CH, CH)], sem_r[p])
                pltpu.async_copy(imag_hbm.at[cidx[p].at[g]],
                                 buf_i[p].at[pl.ds(g * CH, CH)], sem_i[p])

        def wait(p):
            for g in range(ng):
                pltpu.make_async_copy(
                    real_hbm.at[cidx[p].at[g]],
                    buf_r[p].at[pl.ds(g * CH, CH)], sem_r[p]).wait()
                pltpu.make_async_copy(
                    imag_hbm.at[cidx[p].at[g]],
                    buf_i[p].at[pl.ds(g * CH, CH)], sem_i[p]).wait()

        def convert(buf, zb):
            @pl.loop(0, feat)
            def _f(f):
                cf = jnp.full((16,), f, jnp.int32)
                for bb in range(bw // 32):
                    f0 = plsc.load_gather(buf, [ev[bb], cf])
                    f1 = plsc.load_gather(buf, [ev[bb] + 1, cf])
                    zb[f, pl.ds(bb * 32, 32)] = plsc.pack(
                        f0, f1, format=plsc.PackFormat.INTERLEAVED
                    )

        build_cidx(0, 0)
        start(0)

        @pl.loop(0, hist, step=2)
        def _unit(j):
            for p in range(2):
                h = j + p
                q = 1 - p

                @pl.when(h + 1 < hist)
                def _():
                    build_cidx(h + 1, q)
                    start(q)

                wait(p)
                convert(buf_r[p], zb_r)
                convert(buf_i[p], zb_i)
                pltpu.sync_copy(
                    zb_r, out_r_hbm.at[pl.ds(h * feat, feat), pl.ds(b0, bw)])
                pltpu.sync_copy(
                    zb_i, out_i_hbm.at[pl.ds(h * feat, feat), pl.ds(b0, bw)])

    return grab


def kernel(x, real_embed, imag_embed):
    batch, hist = x.shape
    feat = real_embed.shape[1]
    x1 = x.reshape(-1)
    fn = _build(batch, hist, feat)
    z_r, z_i = fn(x1, real_embed, imag_embed)

    def to_logical(z):
        return z.reshape(hist, feat, batch).transpose(2, 0, 1)

    return to_logical(z_r), to_logical(z_i)


# speedup vs baseline: 1.3209x; 1.2593x over previous
import functools

import jax
import jax.numpy as jnp
from jax import lax
from jax.experimental import pallas as pl
from jax.experimental.pallas import tpu as pltpu
from jax.experimental.pallas import tpu_sc as plsc

NC = 2
NS = 16
NW = NC * NS
CH = 128


@functools.lru_cache(maxsize=None)
def _build(batch, hist, feat):
    bw = batch // NW
    ng = bw // CH
    per_w = bw * hist
    mesh = plsc.VectorSubcoreMesh(
        core_axis_name="c", subcore_axis_name="s", num_cores=NC, num_subcores=NS
    )
    out_sds = jax.ShapeDtypeStruct((hist * feat, batch), jnp.bfloat16)
    buf_t = pltpu.VMEM((bw, feat), jnp.float32)
    cidx_t = pltpu.VMEM((ng, CH), jnp.int32)

    @functools.partial(
        pl.kernel,
        out_type=out_sds,
        mesh=mesh,
        compiler_params=pltpu.CompilerParams(
            needs_layout_passes=False, use_tc_tiling_on_sc=False
        ),
        scratch_types=[
            pltpu.VMEM((per_w,), jnp.int32),
            (cidx_t, cidx_t),
            (buf_t, buf_t),
            pltpu.VMEM((feat, bw), jnp.bfloat16),
            (pltpu.SemaphoreType.DMA, pltpu.SemaphoreType.DMA),
        ],
    )
    def grab(x_hbm, tab_hbm, out_hbm, idx_v, cidx, buf, zb, sem):
        wid = lax.axis_index("s") * NC + lax.axis_index("c")
        b0 = wid * bw
        pltpu.sync_copy(x_hbm.at[pl.ds(b0 * hist, per_w)], idx_v)

        i16 = lax.iota(jnp.int32, 16)
        i16h = i16 * hist
        ev = [i16 * 2 + bb * 32 for bb in range(bw // 32)]

        def build_cidx(h, p):
            for g in range(ng):
                for bb in range(CH // 16):
                    base = (g * CH + bb * 16) * hist + h
                    cidx[p][g, pl.ds(bb * 16, 16)] = plsc.load_gather(
                        idx_v, [i16h + base]
                    )

        def start(p):
            for g in range(ng):
                pltpu.async_copy(tab_hbm.at[cidx[p].at[g]],
                                 buf[p].at[pl.ds(g * CH, CH)], sem[p])

        def wait(p):
            for g in range(ng):
                pltpu.make_async_copy(
                    tab_hbm.at[cidx[p].at[g]],
                    buf[p].at[pl.ds(g * CH, CH)], sem[p]).wait()

        def convert(b, zbuf):
            @pl.loop(0, feat)
            def _f(f):
                cf = jnp.full((16,), f, jnp.int32)
                for bb in range(bw // 32):
                    f0 = plsc.load_gather(b, [ev[bb], cf])
                    f1 = plsc.load_gather(b, [ev[bb] + 1, cf])
                    zbuf[f, pl.ds(bb * 32, 32)] = plsc.pack(
                        f0, f1, format=plsc.PackFormat.INTERLEAVED
                    )

        build_cidx(0, 0)
        start(0)

        @pl.loop(0, hist, step=2)
        def _unit(j):
            for p in range(2):
                h = j + p
                q = 1 - p

                @pl.when(h + 1 < hist)
                def _():
                    build_cidx(h + 1, q)
                    start(q)

                wait(p)
                convert(buf[p], zb)
                pltpu.sync_copy(
                    zb, out_hbm.at[pl.ds(h * feat, feat), pl.ds(b0, bw)])

    return grab


def kernel(x, real_embed, imag_embed):
    batch, hist = x.shape
    feat = real_embed.shape[1]
    x1 = x.reshape(-1)
    fn = _build(batch, hist, feat)
    z_r = fn(x1, real_embed)
    z_i = fn(x1, imag_embed)

    def to_logical(z):
        return z.reshape(hist, feat, batch).transpose(2, 0, 1)

    return to_logical(z_r), to_logical(z_i)
